# Initial kernel scaffold; baseline (speedup 1.0000x reference)
import jax
import jax.numpy as jnp
from jax.experimental import pallas as pl


def kernel(psi, u):
    raise NotImplementedError("write your pallas kernel here")


# speedup vs baseline: 4.1650x; 4.1650x over previous
import jax
import jax.numpy as jnp
from jax import lax
from jax.experimental import pallas as pl
from jax.experimental.pallas import tpu as pltpu
from jax.experimental.pallas import tpu_sc as plsc

N = 1 << 24
H = 1 << 23
NC = 2
NS = 16
NW = NC * NS
L = 16

W1 = N // NW
CH1 = 16384
NCH1 = W1 // CH1

G = 2048
R = N // G
RH = R // 2
RPW = RH // NW
CR = 16
NCH2 = RPW // CR

_mesh = plsc.VectorSubcoreMesh(
    core_axis_name="c", subcore_axis_name="s", num_cores=NC, num_subcores=NS
)


def _sums_body(psi_hbm, out_hbm, b0, b1, accv, s0, s1):
    wid = lax.axis_index("s") * NC + lax.axis_index("c")
    base = wid * W1
    bufs = (b0, b1)
    sems = (s0, s1)
    handles = [None, None]
    handles[0] = pltpu.async_copy(psi_hbm.at[pl.ds(base, CH1)], b0, s0)
    acc = jnp.zeros((L,), jnp.float32)
    for g in range(NCH1):
        p = g % 2
        if g + 1 < NCH1:
            handles[1 - p] = pltpu.async_copy(
                psi_hbm.at[pl.ds(base + (g + 1) * CH1, CH1)], bufs[1 - p],
                sems[1 - p])
        handles[p].wait()
        buf = bufs[p]

        def body(j, a):
            x = buf[pl.ds(j * L, L)]
            return a + x * x

        acc = lax.fori_loop(0, CH1 // L, body, acc)
    accv[...] = acc
    pltpu.sync_copy(accv, out_hbm.at[wid])


_sums = pl.kernel(
    _sums_body,
    out_type=jax.ShapeDtypeStruct((NW, L), jnp.float32),
    mesh=_mesh,
    scratch_types=[
        pltpu.VMEM((CH1,), jnp.float32),
        pltpu.VMEM((CH1,), jnp.float32),
        pltpu.VMEM((L,), jnp.float32),
        pltpu.SemaphoreType.DMA,
        pltpu.SemaphoreType.DMA,
    ],
)


def _copy_body(psi_hbm, base_hbm, scale_hbm, out_hbm,
               b0, b1, basev, scalev, s0, s1, o0, o1):
    wid = lax.axis_index("s") * NC + lax.axis_index("c")
    out0 = wid * RPW
    pltpu.sync_copy(base_hbm, basev)
    pltpu.sync_copy(scale_hbm, scalev)
    bvec = basev[...]
    sv = scalev[...]
    iot = lax.iota(jnp.int32, L)
    bufs = (b0, b1)
    gsems = (s0, s1)
    osems = (o0, o1)
    gh = [None, None]
    oh = [None, None]

    def src_idx(g):
        return bvec + (out0 + g * CR) + iot

    gh[0] = pltpu.async_copy(psi_hbm.at[src_idx(0)], b0, s0)
    for g in range(NCH2):
        p = g % 2
        if g + 1 < NCH2:
            if g >= 1:
                oh[1 - p].wait()
            gh[1 - p] = pltpu.async_copy(
                psi_hbm.at[src_idx(g + 1)], bufs[1 - p], gsems[1 - p])
        gh[p].wait()
        buf = bufs[p]

        def mrow(r, _):
            def mcol(j, __):
                buf[r, pl.ds(j * L, L)] = buf[r, pl.ds(j * L, L)] * sv
                return 0

            return lax.fori_loop(0, G // L, mcol, 0)

        lax.fori_loop(0, CR, mrow, 0)
        oh[p] = pltpu.async_copy(buf, out_hbm.at[pl.ds(out0 + g * CR, CR)],
                                 osems[p])
    oh[(NCH2 - 2) % 2].wait()
    oh[(NCH2 - 1) % 2].wait()


_copy = pl.kernel(
    _copy_body,
    out_type=jax.ShapeDtypeStruct((RH, G), jnp.float32),
    mesh=_mesh,
    scratch_types=[
        pltpu.VMEM((CR, G), jnp.float32),
        pltpu.VMEM((CR, G), jnp.float32),
        pltpu.VMEM((L,), jnp.int32),
        pltpu.VMEM((L,), jnp.float32),
        pltpu.SemaphoreType.DMA,
        pltpu.SemaphoreType.DMA,
        pltpu.SemaphoreType.DMA,
        pltpu.SemaphoreType.DMA,
    ],
)


def kernel(psi, u):
    partials = _sums(psi)
    mass0 = jnp.sum(partials[: NW // 2])
    total = jnp.sum(partials)
    p0 = mass0 / total
    outcome = u[0] > p0
    p_out = jnp.where(outcome, 1.0 - p0, p0)
    scale = 1.0 / jnp.sqrt(p_out)
    base = jnp.full((L,), outcome.astype(jnp.int32) * RH, dtype=jnp.int32)
    scale_arr = jnp.full((L,), scale, dtype=jnp.float32)
    out2 = _copy(psi.reshape(R, G), base, scale_arr)
    return out2.reshape(H)


# speedup vs baseline: 13.2551x; 3.1825x over previous
import jax
import jax.numpy as jnp
from jax import lax
from jax.experimental import pallas as pl
from jax.experimental.pallas import tpu as pltpu
from jax.experimental.pallas import tpu_sc as plsc

N = 1 << 24
H = 1 << 23
NC = 2
NS = 16
NW = NC * NS
L = 16

W1 = N // NW
CH1 = 32768
NCH1 = W1 // CH1
UN1 = 8

OPW = H // NW
CH2 = 32768
NCH2 = OPW // CH2
UN2 = 8

_mesh = plsc.VectorSubcoreMesh(
    core_axis_name="c", subcore_axis_name="s", num_cores=NC, num_subcores=NS
)


def _sums_body(psi_hbm, out_hbm, b0, b1, accv, s0, s1):
    wid = lax.axis_index("s") * NC + lax.axis_index("c")
    base = wid * W1
    bufs = (b0, b1)
    sems = (s0, s1)
    handles = [None, None]
    handles[0] = pltpu.async_copy(psi_hbm.at[pl.ds(base, CH1)], b0, s0)
    accs = (jnp.zeros((L,), jnp.float32),) * 4
    for g in range(NCH1):
        p = g % 2
        if g + 1 < NCH1:
            handles[1 - p] = pltpu.async_copy(
                psi_hbm.at[pl.ds(base + (g + 1) * CH1, CH1)], bufs[1 - p],
                sems[1 - p])
        handles[p].wait()
        buf = bufs[p]

        def body(j, a):
            a0, a1, a2, a3 = a
            off = j * (UN1 * L)
            xs = [buf[pl.ds(off + k * L, L)] for k in range(UN1)]
            a0 = a0 + xs[0] * xs[0]
            a1 = a1 + xs[1] * xs[1]
            a2 = a2 + xs[2] * xs[2]
            a3 = a3 + xs[3] * xs[3]
            a0 = a0 + xs[4] * xs[4]
            a1 = a1 + xs[5] * xs[5]
            a2 = a2 + xs[6] * xs[6]
            a3 = a3 + xs[7] * xs[7]
            return (a0, a1, a2, a3)

        accs = lax.fori_loop(0, CH1 // (UN1 * L), body, accs)
    accv[...] = (accs[0] + accs[1]) + (accs[2] + accs[3])
    pltpu.sync_copy(accv, out_hbm.at[wid])


_sums = pl.kernel(
    _sums_body,
    out_type=jax.ShapeDtypeStruct((NW, L), jnp.float32),
    mesh=_mesh,
    scratch_types=[
        pltpu.VMEM((CH1,), jnp.float32),
        pltpu.VMEM((CH1,), jnp.float32),
        pltpu.VMEM((L,), jnp.float32),
        pltpu.SemaphoreType.DMA,
        pltpu.SemaphoreType.DMA,
    ],
)


def _make_copy(half_base):
    def _copy_body(psi_hbm, scale_hbm, out_hbm,
                   b0, b1, scalev, s0, s1, o0, o1):
        wid = lax.axis_index("s") * NC + lax.axis_index("c")
        pltpu.sync_copy(scale_hbm, scalev)
        src0 = half_base + wid * OPW
        dst0 = wid * OPW
        sv = scalev[...]
        bufs = (b0, b1)
        gsems = (s0, s1)
        osems = (o0, o1)
        gh = [None, None]
        oh = [None, None]
        gh[0] = pltpu.async_copy(psi_hbm.at[pl.ds(src0, CH2)], b0, s0)
        for g in range(NCH2):
            p = g % 2
            if g + 1 < NCH2:
                if g >= 1:
                    oh[1 - p].wait()
                gh[1 - p] = pltpu.async_copy(
                    psi_hbm.at[pl.ds(src0 + (g + 1) * CH2, CH2)], bufs[1 - p],
                    gsems[1 - p])
            gh[p].wait()
            buf = bufs[p]

            def mbody(j, _):
                off = j * (UN2 * L)
                for k in range(UN2):
                    buf[pl.ds(off + k * L, L)] = buf[pl.ds(off + k * L, L)] * sv
                return 0

            lax.fori_loop(0, CH2 // (UN2 * L), mbody, 0)
            oh[p] = pltpu.async_copy(
                buf, out_hbm.at[pl.ds(dst0 + g * CH2, CH2)], osems[p])
        oh[(NCH2 - 2) % 2].wait()
        oh[(NCH2 - 1) % 2].wait()

    return pl.kernel(
        _copy_body,
        out_type=jax.ShapeDtypeStruct((H,), jnp.float32),
        mesh=_mesh,
        scratch_types=[
            pltpu.VMEM((CH2,), jnp.float32),
            pltpu.VMEM((CH2,), jnp.float32),
            pltpu.VMEM((L,), jnp.float32),
            pltpu.SemaphoreType.DMA,
            pltpu.SemaphoreType.DMA,
            pltpu.SemaphoreType.DMA,
            pltpu.SemaphoreType.DMA,
        ],
    )


_copy0 = _make_copy(0)
_copy1 = _make_copy(H)


def kernel(psi, u):
    partials = _sums(psi)
    mass0 = jnp.sum(partials[: NW // 2])
    total = jnp.sum(partials)
    p0 = mass0 / total
    outcome = u[0] > p0
    p_out = jnp.where(outcome, 1.0 - p0, p0)
    scale = 1.0 / jnp.sqrt(p_out)
    scale_arr = jnp.full((L,), scale, dtype=jnp.float32)
    return lax.cond(outcome,
                    lambda: _copy1(psi, scale_arr),
                    lambda: _copy0(psi, scale_arr))


# speedup vs baseline: 14.0820x; 1.0624x over previous
import jax
import jax.numpy as jnp
from jax import lax
from jax.experimental import pallas as pl
from jax.experimental.pallas import tpu as pltpu
from jax.experimental.pallas import tpu_sc as plsc

N = 1 << 24
H = 1 << 23
NC = 2
NS = 16
NW = NC * NS
L = 16

W1 = N // NW
CH1 = 32768
NCH1 = W1 // CH1
UN1 = 8

OPW = H // NW
CH2 = 32768
NCH2 = OPW // CH2
UN2 = 8

_mesh = plsc.VectorSubcoreMesh(
    core_axis_name="c", subcore_axis_name="s", num_cores=NC, num_subcores=NS
)


def _sums_body(psi_hbm, out_hbm, b0, b1, b2, accv, s0, s1, s2):
    wid = lax.axis_index("s") * NC + lax.axis_index("c")
    base = wid * W1
    bufs = (b0, b1, b2)
    sems = (s0, s1, s2)
    handles = [None, None, None]
    handles[0] = pltpu.async_copy(psi_hbm.at[pl.ds(base, CH1)], b0, s0)
    handles[1] = pltpu.async_copy(psi_hbm.at[pl.ds(base + CH1, CH1)], b1, s1)
    accs = (jnp.zeros((L,), jnp.float32),) * 4
    for g in range(NCH1):
        p = g % 3
        if g + 2 < NCH1:
            q = (g + 2) % 3
            handles[q] = pltpu.async_copy(
                psi_hbm.at[pl.ds(base + (g + 2) * CH1, CH1)], bufs[q],
                sems[q])
        handles[p].wait()
        buf = bufs[p]

        def body(j, a):
            a0, a1, a2, a3 = a
            off = j * (UN1 * L)
            xs = [buf[pl.ds(off + k * L, L)] for k in range(UN1)]
            a0 = a0 + xs[0] * xs[0]
            a1 = a1 + xs[1] * xs[1]
            a2 = a2 + xs[2] * xs[2]
            a3 = a3 + xs[3] * xs[3]
            a0 = a0 + xs[4] * xs[4]
            a1 = a1 + xs[5] * xs[5]
            a2 = a2 + xs[6] * xs[6]
            a3 = a3 + xs[7] * xs[7]
            return (a0, a1, a2, a3)

        accs = lax.fori_loop(0, CH1 // (UN1 * L), body, accs)
    accv[...] = (accs[0] + accs[1]) + (accs[2] + accs[3])
    pltpu.sync_copy(accv, out_hbm.at[wid])


_sums = pl.kernel(
    _sums_body,
    out_type=jax.ShapeDtypeStruct((NW, L), jnp.float32),
    mesh=_mesh,
    scratch_types=[
        pltpu.VMEM((CH1,), jnp.float32),
        pltpu.VMEM((CH1,), jnp.float32),
        pltpu.VMEM((CH1,), jnp.float32),
        pltpu.VMEM((L,), jnp.float32),
        pltpu.SemaphoreType.DMA,
        pltpu.SemaphoreType.DMA,
        pltpu.SemaphoreType.DMA,
    ],
)


def _make_copy(half_base):
    def _copy_body(psi_hbm, scale_hbm, out_hbm,
                   b0, b1, b2, scalev, s0, s1, s2, o0, o1, o2):
        wid = lax.axis_index("s") * NC + lax.axis_index("c")
        pltpu.sync_copy(scale_hbm, scalev)
        src0 = half_base + wid * OPW
        dst0 = wid * OPW
        sv = scalev[...]
        NB = 3
        bufs = (b0, b1, b2)
        gsems = (s0, s1, s2)
        osems = (o0, o1, o2)
        gh = [None] * NB
        oh = [None] * NB

        def gather(g):
            q = g % NB
            gh[q] = pltpu.async_copy(
                psi_hbm.at[pl.ds(src0 + g * CH2, CH2)], bufs[q], gsems[q])

        for g in range(min(NB - 1, NCH2)):
            gather(g)
        for g in range(NCH2):
            p = g % NB
            nxt = g + NB - 1
            if nxt < NCH2:
                q = nxt % NB
                if oh[q] is not None:
                    oh[q].wait()
                    oh[q] = None
                gather(nxt)
            gh[p].wait()
            buf = bufs[p]

            def mbody(j, _):
                off = j * (UN2 * L)
                for k in range(UN2):
                    buf[pl.ds(off + k * L, L)] = buf[pl.ds(off + k * L, L)] * sv
                return 0

            lax.fori_loop(0, CH2 // (UN2 * L), mbody, 0)
            oh[p] = pltpu.async_copy(
                buf, out_hbm.at[pl.ds(dst0 + g * CH2, CH2)], osems[p])
        for q in range(NB):
            if oh[q] is not None:
                oh[q].wait()

    return pl.kernel(
        _copy_body,
        out_type=jax.ShapeDtypeStruct((H,), jnp.float32),
        mesh=_mesh,
        scratch_types=[
            pltpu.VMEM((CH2,), jnp.float32),
            pltpu.VMEM((CH2,), jnp.float32),
            pltpu.VMEM((CH2,), jnp.float32),
            pltpu.VMEM((L,), jnp.float32),
            pltpu.SemaphoreType.DMA,
            pltpu.SemaphoreType.DMA,
            pltpu.SemaphoreType.DMA,
            pltpu.SemaphoreType.DMA,
            pltpu.SemaphoreType.DMA,
            pltpu.SemaphoreType.DMA,
        ],
    )


_copy0 = _make_copy(0)
_copy1 = _make_copy(H)


def kernel(psi, u):
    partials = _sums(psi)
    mass0 = jnp.sum(partials[: NW // 2])
    total = jnp.sum(partials)
    p0 = mass0 / total
    outcome = u[0] > p0
    p_out = jnp.where(outcome, 1.0 - p0, p0)
    scale = 1.0 / jnp.sqrt(p_out)
    scale_arr = jnp.full((L,), scale, dtype=jnp.float32)
    return lax.cond(outcome,
                    lambda: _copy1(psi, scale_arr),
                    lambda: _copy0(psi, scale_arr))
